# baseline (speedup 1.0000x reference)
import math

import jax
import jax.numpy as jnp
from jax.experimental import pallas as pl
from jax.experimental.pallas import tpu as pltpu

L = 10
BLK = 2048
SUB = BLK // 128


def _sh_block(lonlat_ref, out_ref, sc_ref):
    ll = lonlat_ref[...]
    deg = math.pi / 180.0
    phi = ((ll[:, 0] + 180.0) * deg).reshape(SUB, 128)
    theta = ((ll[:, 1] + 90.0) * deg).reshape(SUB, 128)
    x = jnp.cos(theta)
    sx = jnp.sin(theta)
    cp = jnp.cos(phi)
    sp = jnp.sin(phi)

    cm = jnp.ones_like(x)
    sm = jnp.zeros_like(x)
    pmm = jnp.ones_like(x)

    for m in range(L):
        if m > 0:
            cm, sm = cm * cp - sm * sp, sm * cp + cm * sp
            pmm = (-(2.0 * m - 1.0)) * sx * pmm
        p2 = jnp.zeros_like(x)
        p1 = pmm
        for l in range(m, L):
            if l == m:
                p = pmm
            else:
                p = ((2.0 * l - 1.0) * x * p1 - (l + m - 1.0) * p2) / float(l - m)
                p2, p1 = p1, p
            K = math.sqrt((2.0 * l + 1.0) / (4.0 * math.pi)
                          * math.factorial(l - m) / math.factorial(l + m))
            if m == 0:
                sc_ref[l * l + l] = K * p
            else:
                kp = math.sqrt(2.0) * K
                sc_ref[l * l + l + m] = (kp * cm) * p
                sc_ref[l * l + l - m] = (kp * sm) * p

    for s in range(SUB):
        out_ref[pl.ds(s * 128, 128), :] = sc_ref[:, s, :].T


def kernel(lonlat):
    n = lonlat.shape[0]
    grid = (pl.cdiv(n, BLK),)
    return pl.pallas_call(
        _sh_block,
        grid=grid,
        in_specs=[pl.BlockSpec((BLK, 2), lambda i: (i, 0))],
        out_specs=pl.BlockSpec((BLK, L * L), lambda i: (i, 0)),
        out_shape=jax.ShapeDtypeStruct((n, L * L), jnp.float32),
        scratch_shapes=[pltpu.VMEM((L * L, SUB, 128), jnp.float32)],
    )(lonlat)


# speedup vs baseline: 26.0513x; 26.0513x over previous
import math

import jax
import jax.numpy as jnp
from jax.experimental import pallas as pl
from jax.experimental.pallas import tpu as pltpu

L = 10
BLK = 2048
SUB = BLK // 128


def _sh_block(phi_ref, theta_ref, out_ref, sc_ref):
    phi = phi_ref[...]
    theta = theta_ref[...]
    x = jnp.cos(theta)
    sx = jnp.sin(theta)
    cp = jnp.cos(phi)
    sp = jnp.sin(phi)

    cm = jnp.ones_like(x)
    sm = jnp.zeros_like(x)
    pmm = jnp.ones_like(x)

    for m in range(L):
        if m > 0:
            cm, sm = cm * cp - sm * sp, sm * cp + cm * sp
            pmm = (-(2.0 * m - 1.0)) * sx * pmm
        p2 = jnp.zeros_like(x)
        p1 = pmm
        for l in range(m, L):
            if l == m:
                p = pmm
            else:
                p = ((2.0 * l - 1.0) * x * p1 - (l + m - 1.0) * p2) / float(l - m)
                p2, p1 = p1, p
            K = math.sqrt((2.0 * l + 1.0) / (4.0 * math.pi)
                          * math.factorial(l - m) / math.factorial(l + m))
            if m == 0:
                sc_ref[l * l + l] = K * p
            else:
                kp = math.sqrt(2.0) * K
                sc_ref[l * l + l + m] = (kp * cm) * p
                sc_ref[l * l + l - m] = (kp * sm) * p

    eye = jnp.eye(L * L, dtype=jnp.float32)
    for s in range(SUB):
        tile = sc_ref[:, s, :]
        out_ref[pl.ds(s * 128, 128), :] = jax.lax.dot_general(
            tile, eye,
            dimension_numbers=(((0,), (0,)), ((), ())),
            preferred_element_type=jnp.float32,
            precision=jax.lax.Precision.HIGHEST,
        )


def kernel(lonlat):
    n = lonlat.shape[0]
    nblk = pl.cdiv(n, BLK)
    npad = nblk * BLK
    deg = math.pi / 180.0
    phi = jnp.pad((lonlat[:, 0] + 180.0) * deg, (0, npad - n)).reshape(-1, 128)
    theta = jnp.pad((lonlat[:, 1] + 90.0) * deg, (0, npad - n)).reshape(-1, 128)
    return pl.pallas_call(
        _sh_block,
        grid=(nblk,),
        in_specs=[
            pl.BlockSpec((SUB, 128), lambda i: (i, 0)),
            pl.BlockSpec((SUB, 128), lambda i: (i, 0)),
        ],
        out_specs=pl.BlockSpec((BLK, L * L), lambda i: (i, 0)),
        out_shape=jax.ShapeDtypeStruct((n, L * L), jnp.float32),
        scratch_shapes=[pltpu.VMEM((L * L, SUB, 128), jnp.float32)],
    )(phi, theta)


# speedup vs baseline: 26.3948x; 1.0132x over previous
import math

import jax
import jax.numpy as jnp
from jax.experimental import pallas as pl
from jax.experimental.pallas import tpu as pltpu

L = 10
BLK = 2048
SUB = BLK // 128


def _K(l, m):
    return math.sqrt((2.0 * l + 1.0) / (4.0 * math.pi)
                     * math.factorial(l - m) / math.factorial(l + m))


def _sinpoly(z):
    w = z * z
    s = 1.5918144e-10
    for c in (-2.5052108e-08, 2.7557319e-06, -1.9841270e-04,
              8.3333333e-03, -1.6666667e-01, 1.0):
        s = s * w + c
    return s * z


def _cospoly(z):
    w = z * z
    s = -1.1470746e-11
    for c in (2.0876757e-09, -2.7557319e-07, 2.4801587e-05,
              -1.3888889e-03, 4.1666667e-02, -5.0e-01, 1.0):
        s = s * w + c
    return s


def _sh_block(lonh_ref, latn_ref, out_ref, sc_ref):
    lonh = lonh_ref[...]
    latn = latn_ref[...]
    x = _sinpoly(latn)
    sx = _cospoly(latn)
    sh = _sinpoly(lonh)
    ch = _cospoly(lonh)
    cp = 2.0 * sh * sh - 1.0
    sp = -2.0 * sh * ch

    cm = jnp.ones_like(x)
    sm = jnp.zeros_like(x)
    pmm = jnp.full_like(x, _K(0, 0))

    for m in range(L):
        if m > 0:
            cm, sm = cm * cp - sm * sp, sm * cp + cm * sp
            f = -(2.0 * m - 1.0) * _K(m, m) / _K(m - 1, m - 1)
            if m == 1:
                f *= math.sqrt(2.0)
            pmm = f * sx * pmm
        p2 = jnp.zeros_like(x)
        p1 = pmm
        for l in range(m, L):
            if l == m:
                p = pmm
            else:
                a = _K(l, m) / _K(l - 1, m) * (2.0 * l - 1.0) / float(l - m)
                b = (-_K(l, m) / _K(l - 2, m) * (l + m - 1.0) / float(l - m)
                     if l >= m + 2 else 0.0)
                p = a * (x * p1) + b * p2
                p2, p1 = p1, p
            if m == 0:
                sc_ref[l * l + l] = p
            else:
                sc_ref[l * l + l + m] = cm * p
                sc_ref[l * l + l - m] = sm * p

    eye = jnp.eye(L * L, dtype=jnp.float32)
    for s in range(SUB):
        out_ref[pl.ds(s * 128, 128), :] = jax.lax.dot_general(
            sc_ref[:, s, :], eye,
            dimension_numbers=(((0,), (0,)), ((), ())),
            preferred_element_type=jnp.float32,
            precision=jax.lax.Precision.HIGHEST,
        )


def kernel(lonlat):
    n = lonlat.shape[0]
    nblk = pl.cdiv(n, BLK)
    npad = nblk * BLK
    lonh = jnp.pad(lonlat[:, 0] * (math.pi / 360.0), (0, npad - n)).reshape(-1, 128)
    latn = jnp.pad(lonlat[:, 1] * (-math.pi / 180.0), (0, npad - n)).reshape(-1, 128)
    return pl.pallas_call(
        _sh_block,
        grid=(nblk,),
        in_specs=[
            pl.BlockSpec((SUB, 128), lambda i: (i, 0)),
            pl.BlockSpec((SUB, 128), lambda i: (i, 0)),
        ],
        out_specs=pl.BlockSpec((BLK, L * L), lambda i: (i, 0)),
        out_shape=jax.ShapeDtypeStruct((n, L * L), jnp.float32),
        scratch_shapes=[pltpu.VMEM((L * L, SUB, 128), jnp.float32)],
    )(lonh, latn)


# speedup vs baseline: 32.2780x; 1.2229x over previous
import math

import jax
import jax.numpy as jnp
from jax.experimental import pallas as pl
from jax.experimental.pallas import tpu as pltpu

L = 10
BLK = 2048
SUB = BLK // 128


def _K(l, m):
    return math.sqrt((2.0 * l + 1.0) / (4.0 * math.pi)
                     * math.factorial(l - m) / math.factorial(l + m))


def _sinpoly(z):
    w = z * z
    s = 1.5918144e-10
    for c in (-2.5052108e-08, 2.7557319e-06, -1.9841270e-04,
              8.3333333e-03, -1.6666667e-01, 1.0):
        s = s * w + c
    return s * z


def _cospoly(z):
    w = z * z
    s = -1.1470746e-11
    for c in (2.0876757e-09, -2.7557319e-07, 2.4801587e-05,
              -1.3888889e-03, 4.1666667e-02, -5.0e-01, 1.0):
        s = s * w + c
    return s


def _sh_block(lonh_ref, latn_ref, out_ref, sc_ref):
    lonh = lonh_ref[...]
    latn = latn_ref[...]
    x = _sinpoly(latn)
    sx = _cospoly(latn)
    sh = _sinpoly(lonh)
    ch = _cospoly(lonh)
    cp = 2.0 * sh * sh - 1.0
    sp = -2.0 * sh * ch

    cm = jnp.ones_like(x)
    sm = jnp.zeros_like(x)
    pmm = jnp.full_like(x, _K(0, 0))

    for m in range(L):
        if m > 0:
            cm, sm = cm * cp - sm * sp, sm * cp + cm * sp
            f = -(2.0 * m - 1.0) * _K(m, m) / _K(m - 1, m - 1)
            if m == 1:
                f *= math.sqrt(2.0)
            pmm = f * sx * pmm
        p2 = jnp.zeros_like(x)
        p1 = pmm
        for l in range(m, L):
            if l == m:
                p = pmm
            else:
                a = _K(l, m) / _K(l - 1, m) * (2.0 * l - 1.0) / float(l - m)
                b = (-_K(l, m) / _K(l - 2, m) * (l + m - 1.0) / float(l - m)
                     if l >= m + 2 else 0.0)
                p = a * (x * p1) + b * p2
                p2, p1 = p1, p
            if m == 0:
                sc_ref[l * l + l] = p
            else:
                sc_ref[l * l + l + m] = cm * p
                sc_ref[l * l + l - m] = sm * p

    eye = jnp.eye(L * L, dtype=jnp.float32)
    for s in range(SUB):
        out_ref[pl.ds(s * 128, 128), :] = jax.lax.dot_general(
            sc_ref[:, s, :], eye,
            dimension_numbers=(((0,), (0,)), ((), ())),
            preferred_element_type=jnp.float32,
            precision=jax.lax.Precision.DEFAULT,
        )


def kernel(lonlat):
    n = lonlat.shape[0]
    nblk = pl.cdiv(n, BLK)
    npad = nblk * BLK
    lonh = jnp.pad(lonlat[:, 0] * (math.pi / 360.0), (0, npad - n)).reshape(-1, 128)
    latn = jnp.pad(lonlat[:, 1] * (-math.pi / 180.0), (0, npad - n)).reshape(-1, 128)
    return pl.pallas_call(
        _sh_block,
        grid=(nblk,),
        in_specs=[
            pl.BlockSpec((SUB, 128), lambda i: (i, 0)),
            pl.BlockSpec((SUB, 128), lambda i: (i, 0)),
        ],
        out_specs=pl.BlockSpec((BLK, L * L), lambda i: (i, 0)),
        out_shape=jax.ShapeDtypeStruct((n, L * L), jnp.float32),
        scratch_shapes=[pltpu.VMEM((L * L, SUB, 128), jnp.float32)],
    )(lonh, latn)
